# baseline (speedup 1.0000x reference)
import functools

import jax
import jax.numpy as jnp
from jax import lax
from jax.experimental import pallas as pl
from jax.experimental.pallas import tpu as pltpu
from jax.experimental.pallas import tpu_sc as plsc

N_NODES = 10000
N_PAD = 10240
IN_DIM = 128
HID = 32
OUT = 7
D2 = 16

NC, NS, L = 2, 16, 16
NW = NC * NS
N_EDGES = 320000
E_PAD = 327680
CHUNK = 80
EPW = E_PAD // NW
NCHUNK = EPW // CHUNK
NBUF = 8
ROWS_PER_TILE = N_PAD // NS
ZCOPIES = ROWS_PER_TILE // CHUNK


def _make_spmm(d):
    mesh = plsc.VectorSubcoreMesh(
        core_axis_name="c", subcore_axis_name="s",
        num_cores=NC, num_subcores=NS)

    dn = lax.GatherDimensionNumbers(
        offset_dims=(), collapsed_slice_dims=(0,), start_index_map=(0,))
    nvec = d // L

    @functools.partial(
        pl.kernel,
        out_type=jax.ShapeDtypeStruct((NC, N_PAD, d), jnp.float32),
        mesh=mesh,
        compiler_params=pltpu.CompilerParams(use_tc_tiling_on_sc=False),
        scratch_types=[
            pltpu.VMEM((NCHUNK, CHUNK), jnp.int32),
            pltpu.VMEM((NCHUNK, CHUNK), jnp.int32),
            pltpu.VMEM((EPW,), jnp.float32),
            pltpu.VMEM((NBUF, CHUNK, d), jnp.float32),
            pltpu.VMEM((NBUF, CHUNK, d), jnp.float32),
            pltpu.VMEM_SHARED((N_PAD, d), jnp.float32),
            pltpu.SemaphoreType.DMA((NBUF,)),
            pltpu.SemaphoreType.DMA((NBUF,)),
        ],
    )
    def spmm(z_hbm, ei_hbm, w_hbm, out_hbm,
             src_v, dst_v, w_v, ring_a, ring_b, acc_sh, semg, sems):
        c = lax.axis_index("c")
        s = lax.axis_index("s")
        wid = c * NS + s
        e0 = pl.multiple_of(wid * EPW, 8)
        pltpu.sync_copy(ei_hbm.at[0, wid], src_v)
        pltpu.sync_copy(ei_hbm.at[1, wid], dst_v)
        pltpu.sync_copy(w_hbm.at[pl.ds(e0, EPW)], w_v)

        zv = jnp.zeros((L,), jnp.float32)
        for b in range(NBUF):
            def zbody(e, carry):
                for h in range(nvec):
                    ring_b[b, e, h * L:(h + 1) * L] = zv
                return carry
            lax.fori_loop(0, CHUNK, zbody, 0)
        n0 = s * ROWS_PER_TILE
        for t in range(ZCOPIES):
            pltpu.sync_copy(ring_b.at[0],
                            acc_sh.at[pl.ds(n0 + t * CHUNK, CHUNK)])
        plsc.subcore_barrier()

        for b in range(NBUF):
            pltpu.async_copy(ring_b.at[b], acc_sh.at[dst_v.at[b]],
                             sems.at[b], add=True)
            pltpu.async_copy(z_hbm.at[src_v.at[b]], ring_a.at[b], semg.at[b])

        def round_body(g, carry):
            for b in range(NBUF):
                j = g * NBUF + b
                jc = pl.multiple_of(j * CHUNK, 8)
                pltpu.make_async_copy(
                    z_hbm.at[src_v.at[b]], ring_a.at[b], semg.at[b]).wait()
                pltpu.make_async_copy(
                    ring_b.at[b], acc_sh.at[dst_v.at[b]], sems.at[b]).wait()

                def scale_body(grp, carry2):
                    w16 = w_v[pl.ds(jc + grp * L, L)]
                    eb = grp * L
                    for l in range(L):
                        wv = lax.gather(
                            w16, jnp.full((L, 1), l, jnp.int32), dn, (1,),
                            mode=lax.GatherScatterMode.PROMISE_IN_BOUNDS)
                        for h in range(nvec):
                            ring_b[b, eb + l, h * L:(h + 1) * L] = (
                                ring_a[b, eb + l, h * L:(h + 1) * L] * wv)
                    return carry2

                lax.fori_loop(0, CHUNK // L, scale_body, 0)
                jn = jnp.minimum(j + NBUF, NCHUNK - 1)
                pltpu.async_copy(z_hbm.at[src_v.at[jn]], ring_a.at[b],
                                 semg.at[b])
                pltpu.async_copy(ring_b.at[b], acc_sh.at[dst_v.at[j]],
                                 sems.at[b], add=True)
            return carry

        lax.fori_loop(0, NCHUNK // NBUF, round_body, 0)
        for b in range(NBUF):
            pltpu.make_async_copy(
                z_hbm.at[src_v.at[b]], ring_a.at[b], semg.at[b]).wait()
            pltpu.make_async_copy(
                ring_b.at[b], acc_sh.at[dst_v.at[b]], sems.at[b]).wait()
        plsc.subcore_barrier()
        pltpu.sync_copy(acc_sh.at[pl.ds(n0, ROWS_PER_TILE)],
                        out_hbm.at[c, pl.ds(n0, ROWS_PER_TILE)])

    return spmm


_spmm1 = _make_spmm(HID)

_PACK = 128 // HID
N_PK = N_PAD // _PACK


def _mm1_body(x_ref, w_ref, o_ref):
    o_ref[...] = jnp.dot(x_ref[...], w_ref[...],
                         preferred_element_type=jnp.float32)


def _mid_body(p_ref, b1_ref, w2_ref, o_ref):
    h = jnp.maximum(p_ref[0] + p_ref[1] + b1_ref[...], 0.0)
    o_ref[...] = jnp.dot(h, w2_ref[...], preferred_element_type=jnp.float32)


def _fin_body(q_ref, b2_ref, o_ref):
    o_ref[...] = q_ref[0] + q_ref[1] + b2_ref[...]


_RB = 1000
_RBK = 256


def kernel(edge_index, edge_weight, x, W1, b1, W2, b2):
    ei4 = jnp.pad(edge_index.astype(jnp.int32),
                  ((0, 0), (0, E_PAD - N_EDGES))).reshape(
                      2, NW, NCHUNK, CHUNK)
    wpad = jnp.pad(edge_weight, (0, E_PAD - N_EDGES))

    z = pl.pallas_call(
        _mm1_body,
        grid=(N_NODES // _RB,),
        in_specs=[pl.BlockSpec((_RB, IN_DIM), lambda i: (i, 0)),
                  pl.BlockSpec((IN_DIM, HID), lambda i: (0, 0))],
        out_specs=pl.BlockSpec((_RB, HID), lambda i: (i, 0)),
        out_shape=jax.ShapeDtypeStruct((N_NODES, HID), jnp.float32),
    )(x, W1)

    p = _spmm1(z, ei4, wpad)

    p128 = p.reshape(NC, N_PK, 128)
    w2p = jnp.pad(W2, ((0, 0), (0, HID - OUT)))
    w2bd = jnp.kron(jnp.eye(_PACK, dtype=jnp.float32), w2p)
    b1t = jnp.tile(b1, _PACK).reshape(1, 128)
    u128 = pl.pallas_call(
        _mid_body,
        grid=(N_PK // _RBK,),
        in_specs=[pl.BlockSpec((NC, _RBK, 128), lambda i: (0, i, 0)),
                  pl.BlockSpec((1, 128), lambda i: (0, 0)),
                  pl.BlockSpec((128, 128), lambda i: (0, 0))],
        out_specs=pl.BlockSpec((_RBK, 128), lambda i: (i, 0)),
        out_shape=jax.ShapeDtypeStruct((N_PK, 128), jnp.float32),
    )(p128, b1t, w2bd)

    q = _spmm1(u128.reshape(N_PAD, HID), ei4, wpad)

    b2t = jnp.tile(jnp.pad(b2, (0, HID - OUT)), _PACK).reshape(1, 128)
    out128 = pl.pallas_call(
        _fin_body,
        grid=(N_PK // _RBK,),
        in_specs=[pl.BlockSpec((NC, _RBK, 128), lambda i: (0, i, 0)),
                  pl.BlockSpec((1, 128), lambda i: (0, 0))],
        out_specs=pl.BlockSpec((_RBK, 128), lambda i: (i, 0)),
        out_shape=jax.ShapeDtypeStruct((N_PK, 128), jnp.float32),
    )(q.reshape(NC, N_PK, 128), b2t)

    return out128.reshape(N_PAD, HID)[:N_NODES, :OUT]


# speedup vs baseline: 2.4544x; 2.4544x over previous
import functools

import jax
import jax.numpy as jnp
from jax import lax
from jax.experimental import pallas as pl
from jax.experimental.pallas import tpu as pltpu
from jax.experimental.pallas import tpu_sc as plsc

N_NODES = 10000
N_PAD = 10240
IN_DIM = 128
HID = 32
OUT = 7
D2 = 16

NC, NS, L = 2, 16, 16
NW = NC * NS
N_EDGES = 320000
CHUNK = 80
EPW = N_EDGES // NW
NCHUNK = EPW // CHUNK
NBUF = 5
ROWS_PER_TILE = N_PAD // NS
ZCOPIES = ROWS_PER_TILE // CHUNK


def _make_spmm(d):
    mesh = plsc.VectorSubcoreMesh(
        core_axis_name="c", subcore_axis_name="s",
        num_cores=NC, num_subcores=NS)

    dn = lax.GatherDimensionNumbers(
        offset_dims=(), collapsed_slice_dims=(0,), start_index_map=(0,))
    nvec = d // L

    @functools.partial(
        pl.kernel,
        out_type=jax.ShapeDtypeStruct((NC, N_PAD, d), jnp.float32),
        mesh=mesh,
        compiler_params=pltpu.CompilerParams(use_tc_tiling_on_sc=False),
        scratch_types=[
            pltpu.VMEM((NCHUNK, CHUNK), jnp.int32),
            pltpu.VMEM((NCHUNK, CHUNK), jnp.int32),
            pltpu.VMEM((EPW,), jnp.float32),
            pltpu.VMEM((NBUF, CHUNK, d), jnp.float32),
            pltpu.VMEM((NBUF, CHUNK, d), jnp.float32),
            pltpu.VMEM_SHARED((N_PAD, d), jnp.float32),
            pltpu.SemaphoreType.DMA((NBUF,)),
            pltpu.SemaphoreType.DMA((NBUF,)),
        ],
    )
    def spmm(z_hbm, ei_hbm, w_hbm, out_hbm,
             src_v, dst_v, w_v, ring_a, ring_b, acc_sh, semg, sems):
        c = lax.axis_index("c")
        s = lax.axis_index("s")
        wid = c * NS + s
        e0 = pl.multiple_of(wid * EPW, 8)
        pltpu.sync_copy(ei_hbm.at[0, wid], src_v)
        pltpu.sync_copy(ei_hbm.at[1, wid], dst_v)
        pltpu.sync_copy(w_hbm.at[pl.ds(e0, EPW)], w_v)

        zv = jnp.zeros((L,), jnp.float32)
        for b in range(NBUF):
            def zbody(e, carry):
                for h in range(nvec):
                    ring_b[b, e, h * L:(h + 1) * L] = zv
                return carry
            lax.fori_loop(0, CHUNK, zbody, 0)
        n0 = s * ROWS_PER_TILE
        for t in range(ZCOPIES):
            pltpu.sync_copy(ring_b.at[0],
                            acc_sh.at[pl.ds(n0 + t * CHUNK, CHUNK)])
        plsc.subcore_barrier()

        for b in range(NBUF):
            pltpu.async_copy(ring_b.at[b], acc_sh.at[dst_v.at[b]],
                             sems.at[b], add=True)
            pltpu.async_copy(z_hbm.at[src_v.at[b]], ring_a.at[b], semg.at[b])

        def round_body(g, carry):
            for b in range(NBUF):
                j = g * NBUF + b
                jc = pl.multiple_of(j * CHUNK, 8)
                pltpu.make_async_copy(
                    z_hbm.at[src_v.at[b]], ring_a.at[b], semg.at[b]).wait()
                pltpu.make_async_copy(
                    ring_b.at[b], acc_sh.at[dst_v.at[b]], sems.at[b]).wait()

                def scale_body(grp, carry2):
                    w16 = w_v[pl.ds(jc + grp * L, L)]
                    eb = grp * L
                    for l in range(L):
                        wv = lax.gather(
                            w16, jnp.full((L, 1), l, jnp.int32), dn, (1,),
                            mode=lax.GatherScatterMode.PROMISE_IN_BOUNDS)
                        for h in range(nvec):
                            ring_b[b, eb + l, h * L:(h + 1) * L] = (
                                ring_a[b, eb + l, h * L:(h + 1) * L] * wv)
                    return carry2

                lax.fori_loop(0, CHUNK // L, scale_body, 0)
                jn = jnp.minimum(j + NBUF, NCHUNK - 1)
                pltpu.async_copy(z_hbm.at[src_v.at[jn]], ring_a.at[b],
                                 semg.at[b])
                pltpu.async_copy(ring_b.at[b], acc_sh.at[dst_v.at[j]],
                                 sems.at[b], add=True)
            return carry

        lax.fori_loop(0, NCHUNK // NBUF, round_body, 0)
        for b in range(NBUF):
            pltpu.make_async_copy(
                z_hbm.at[src_v.at[b]], ring_a.at[b], semg.at[b]).wait()
            pltpu.make_async_copy(
                ring_b.at[b], acc_sh.at[dst_v.at[b]], sems.at[b]).wait()
        plsc.subcore_barrier()
        pltpu.sync_copy(acc_sh.at[pl.ds(n0, ROWS_PER_TILE)],
                        out_hbm.at[c, pl.ds(n0, ROWS_PER_TILE)])

    return spmm


_spmm1 = _make_spmm(HID)

_PACK = 128 // HID
N_PK = N_PAD // _PACK


def _mm1_body(x_ref, w_ref, o_ref):
    o_ref[...] = jnp.dot(x_ref[...], w_ref[...],
                         preferred_element_type=jnp.float32)


def _mid_body(p_ref, b1_ref, w2_ref, o_ref):
    h = jnp.maximum(p_ref[0] + p_ref[1] + b1_ref[...], 0.0)
    o_ref[...] = jnp.dot(h, w2_ref[...], preferred_element_type=jnp.float32)


def _fin_body(q_ref, b2_ref, o_ref):
    o_ref[...] = q_ref[0] + q_ref[1] + b2_ref[...]


_RB = 1000
_RBK = 256


def kernel(edge_index, edge_weight, x, W1, b1, W2, b2):
    ei4 = edge_index.astype(jnp.int32).reshape(2, NW, NCHUNK, CHUNK)

    z = pl.pallas_call(
        _mm1_body,
        grid=(N_NODES // _RB,),
        in_specs=[pl.BlockSpec((_RB, IN_DIM), lambda i: (i, 0)),
                  pl.BlockSpec((IN_DIM, HID), lambda i: (0, 0))],
        out_specs=pl.BlockSpec((_RB, HID), lambda i: (i, 0)),
        out_shape=jax.ShapeDtypeStruct((N_NODES, HID), jnp.float32),
    )(x, W1)

    p = _spmm1(z, ei4, edge_weight)

    p128 = p.reshape(NC, N_PK, 128)
    w2p = jnp.pad(W2, ((0, 0), (0, HID - OUT)))
    w2bd = jnp.kron(jnp.eye(_PACK, dtype=jnp.float32), w2p)
    b1t = jnp.tile(b1, _PACK).reshape(1, 128)
    u128 = pl.pallas_call(
        _mid_body,
        grid=(N_PK // _RBK,),
        in_specs=[pl.BlockSpec((NC, _RBK, 128), lambda i: (0, i, 0)),
                  pl.BlockSpec((1, 128), lambda i: (0, 0)),
                  pl.BlockSpec((128, 128), lambda i: (0, 0))],
        out_specs=pl.BlockSpec((_RBK, 128), lambda i: (i, 0)),
        out_shape=jax.ShapeDtypeStruct((N_PK, 128), jnp.float32),
    )(p128, b1t, w2bd)

    q = _spmm1(u128.reshape(N_PAD, HID), ei4, edge_weight)

    b2t = jnp.tile(jnp.pad(b2, (0, HID - OUT)), _PACK).reshape(1, 128)
    out128 = pl.pallas_call(
        _fin_body,
        grid=(N_PK // _RBK,),
        in_specs=[pl.BlockSpec((NC, _RBK, 128), lambda i: (0, i, 0)),
                  pl.BlockSpec((1, 128), lambda i: (0, 0))],
        out_specs=pl.BlockSpec((_RBK, 128), lambda i: (i, 0)),
        out_shape=jax.ShapeDtypeStruct((N_PK, 128), jnp.float32),
    )(q.reshape(NC, N_PK, 128), b2t)

    return out128.reshape(N_PAD, HID)[:N_NODES, :OUT]
